# Initial kernel scaffold; baseline (speedup 1.0000x reference)
import jax
import jax.numpy as jnp
from jax.experimental import pallas as pl


def kernel(node_features, edge_index, edge_features, W1, b1, W2, b2, Wu1, bu1, Wu2, bu2, gamma, beta):
    raise NotImplementedError("write your pallas kernel here")


# speedup vs baseline: 1.5709x; 1.5709x over previous
import functools

import jax
import jax.numpy as jnp
from jax import lax
from jax.experimental import pallas as pl
from jax.experimental.pallas import tpu as pltpu
from jax.experimental.pallas import tpu_sc as plsc

N = 10000
E = 160000
D = 256
DE = 16
H = 512
GC = 128
NG = H // GC
CHUNK = 128
NCH = E // CHUNK
NWORK = 32
FLUSH_ROWS = 624
FLUSH_TAIL = N - 16 * FLUSH_ROWS


def _proj_body(x_ref, w_ref, out_ref):
    x = x_ref[...]
    for t in range(2 * NG):
        r0 = D * (t // NG)
        c0 = GC * (t % NG)
        out_ref[t] = jnp.dot(x, w_ref[r0:r0 + D, c0:c0 + GC],
                             preferred_element_type=jnp.float32)


def _tc_proj(x, w1):
    nb = 2000
    return pl.pallas_call(
        _proj_body,
        grid=(N // nb,),
        in_specs=[
            pl.BlockSpec((nb, D), lambda n: (n, 0)),
            pl.BlockSpec((2 * D + DE, H), lambda n: (0, 0)),
        ],
        out_specs=pl.BlockSpec((2 * NG, nb, GC), lambda n: (0, n, 0)),
        out_shape=jax.ShapeDtypeStruct((2 * NG, N, GC), jnp.float32),
    )(x, w1)


def _gather_body(p8, idx_h, out, idxb, gbuf, gsem):
    c = lax.axis_index("c")
    s = lax.axis_index("s")
    wid = s * 2 + c

    def iter_body(i, carry):
        r = wid + i * NWORK

        @pl.when(r < NCH)
        def _():
            for t in range(2 * NG):
                pltpu.sync_copy(idx_h.at[pl.ds(t * E + r * CHUNK, CHUNK)],
                                idxb)
                pltpu.async_copy(p8.at[idxb], gbuf, gsem).wait()
                pltpu.sync_copy(gbuf, out.at[pl.ds(t * E + r * CHUNK, CHUNK)])

        return carry

    lax.fori_loop(0, NCH // NWORK + 1, iter_body, 0)


def _sc_gather(p8_flat, idx_all):
    mesh = plsc.VectorSubcoreMesh(core_axis_name="c", subcore_axis_name="s")
    f = pl.kernel(
        _gather_body,
        out_type=jax.ShapeDtypeStruct((2 * NG * E, GC), jnp.float32),
        mesh=mesh,
        scratch_types=[
            pltpu.VMEM((CHUNK,), jnp.int32),
            pltpu.VMEM((CHUNK, GC), jnp.float32),
            pltpu.SemaphoreType.DMA,
        ],
    )
    return f(p8_flat, idx_all)


_INV_SQRT2 = 0.7071067811865476


def _gelu_body(g8_ref, ef_ref, wc_ref, b1_ref, out_ref):
    ef = ef_ref[...]
    for g in range(NG):
        z = (g8_ref[g] + g8_ref[NG + g]
             + jnp.dot(ef, wc_ref[:, g * GC:(g + 1) * GC],
                       preferred_element_type=jnp.float32)
             + b1_ref[0, g * GC:(g + 1) * GC][None, :])
        out_ref[g] = z * 0.5 * (1.0 + lax.erf(z * _INV_SQRT2))


def _tc_gelu(g8, ef, wc, b1_2d):
    be = 1000
    return pl.pallas_call(
        _gelu_body,
        grid=(E // be,),
        in_specs=[
            pl.BlockSpec((2 * NG, be, GC), lambda e: (0, e, 0)),
            pl.BlockSpec((be, DE), lambda e: (e, 0)),
            pl.BlockSpec((DE, H), lambda e: (0, 0)),
            pl.BlockSpec((1, H), lambda e: (0, 0)),
        ],
        out_specs=pl.BlockSpec((NG, be, GC), lambda e: (0, e, 0)),
        out_shape=jax.ShapeDtypeStruct((NG, E, GC), jnp.float32),
    )(g8, ef, wc, b1_2d)


def _scatter_body(hm, dst_h, s_out, deg_out, acc, hmbuf, idx1, sem):
    c = lax.axis_index("c")
    s = lax.axis_index("s")
    base = s * FLUSH_ROWS
    zero_chunks = [(0, CHUNK), (CHUNK, CHUNK), (2 * CHUNK, CHUNK),
                   (3 * CHUNK, CHUNK), (4 * CHUNK, FLUSH_ROWS - 4 * CHUNK)]

    def fill_hmbuf(val):
        def fill(j, carry):
            hmbuf[j // 8, pl.ds((j % 8) * 16, 16)] = jnp.full((16,), val,
                                                              jnp.float32)
            return carry
        lax.fori_loop(0, CHUNK * (GC // 16), fill, 0)

    def zero_acc():
        fill_hmbuf(0.0)
        for off, ln in zero_chunks:
            pltpu.sync_copy(hmbuf.at[pl.ds(0, ln)],
                            acc.at[pl.ds(base + off, ln)])

        @pl.when(s == 0)
        def _():
            pltpu.sync_copy(hmbuf.at[pl.ds(0, FLUSH_TAIL)],
                            acc.at[pl.ds(16 * FLUSH_ROWS, FLUSH_TAIL)])

    def flush_acc(out_ref, row0):
        pltpu.sync_copy(acc.at[pl.ds(base, FLUSH_ROWS)],
                        out_ref.at[pl.ds(row0 + base, FLUSH_ROWS)])

        @pl.when(s == 0)
        def _():
            pltpu.sync_copy(
                acc.at[pl.ds(16 * FLUSH_ROWS, FLUSH_TAIL)],
                out_ref.at[pl.ds(row0 + 16 * FLUSH_ROWS, FLUSH_TAIL)])

    def edge_loop(body):
        def iter_body(i, carry):
            r = s + i * 16

            @pl.when(r < NCH)
            def _():
                body(r)

            return carry

        lax.fori_loop(0, NCH // 16 + 1, iter_body, 0)

    for gg in range(2):
        g = c * 2 + gg
        zero_acc()
        plsc.subcore_barrier()

        def scat(r):
            pltpu.sync_copy(dst_h.at[pl.ds(r * CHUNK, CHUNK)], idx1)
            pltpu.sync_copy(hm.at[pl.ds(g * E + r * CHUNK, CHUNK)], hmbuf)
            pltpu.sync_copy(hmbuf, acc.at[idx1], add=True)

        edge_loop(scat)
        plsc.subcore_barrier()
        flush_acc(s_out, g * N)
        plsc.subcore_barrier()

    @pl.when(c == 0)
    def _():
        zero_acc()
    plsc.subcore_barrier()

    @pl.when(c == 0)
    def _():
        fill_hmbuf(1.0)

        def scat_deg(r):
            pltpu.sync_copy(dst_h.at[pl.ds(r * CHUNK, CHUNK)], idx1)
            pltpu.sync_copy(hmbuf, acc.at[idx1], add=True)

        edge_loop(scat_deg)
    plsc.subcore_barrier()

    @pl.when(c == 0)
    def _():
        flush_acc(deg_out, 0)


def _sc_scatter(hm_flat, dst):
    mesh = plsc.VectorSubcoreMesh(core_axis_name="c", subcore_axis_name="s")
    f = pl.kernel(
        _scatter_body,
        out_type=[
            jax.ShapeDtypeStruct((NG * N, GC), jnp.float32),
            jax.ShapeDtypeStruct((N, GC), jnp.float32),
        ],
        mesh=mesh,
        scratch_types=[
            pltpu.VMEM_SHARED((N, GC), jnp.float32),
            pltpu.VMEM((CHUNK, GC), jnp.float32),
            pltpu.VMEM((CHUNK,), jnp.int32),
            pltpu.SemaphoreType.DMA,
        ],
    )
    return f(hm_flat, dst)


def _update_body(x_ref, s_ref, deg_ref, w2_ref, b2_ref, wu1_ref, bu1_ref,
                 wu2_ref, bu2_ref, gamma_ref, beta_ref, out_ref):
    x = x_ref[...]
    sw = jnp.dot(s_ref[0], w2_ref[0:GC, :], preferred_element_type=jnp.float32)
    for g in range(1, NG):
        sw = sw + jnp.dot(s_ref[g], w2_ref[g * GC:(g + 1) * GC, :],
                          preferred_element_type=jnp.float32)
    deg = deg_ref[:, 0:1]
    md = jnp.maximum(deg, 1.0)
    agg = sw / md + (deg / md) * b2_ref[...]
    t = (jnp.dot(x, wu1_ref[0:D, :], preferred_element_type=jnp.float32)
         + jnp.dot(agg, wu1_ref[D:2 * D, :], preferred_element_type=jnp.float32)
         + bu1_ref[...])
    u = t * 0.5 * (1.0 + lax.erf(t * _INV_SQRT2))
    y = (jnp.dot(u, wu2_ref[...], preferred_element_type=jnp.float32)
         + bu2_ref[...] + x)
    mu = jnp.mean(y, axis=-1, keepdims=True)
    d = y - mu
    var = jnp.mean(d * d, axis=-1, keepdims=True)
    out_ref[...] = d * lax.rsqrt(var + 1e-5) * gamma_ref[...] + beta_ref[...]


def _tc_update(x, s4, deg, w2, b2, wu1, bu1, wu2, bu2, gamma, beta):
    nb = 1000
    full = lambda shape: pl.BlockSpec(shape, lambda n: tuple(0 for _ in shape))
    return pl.pallas_call(
        _update_body,
        grid=(N // nb,),
        in_specs=[
            pl.BlockSpec((nb, D), lambda n: (n, 0)),
            pl.BlockSpec((NG, nb, GC), lambda n: (0, n, 0)),
            pl.BlockSpec((nb, GC), lambda n: (n, 0)),
            full((H, D)),
            full((1, D)),
            full((2 * D, H)),
            full((1, H)),
            full((H, D)),
            full((1, D)),
            full((1, D)),
            full((1, D)),
        ],
        out_specs=pl.BlockSpec((nb, D), lambda n: (n, 0)),
        out_shape=jax.ShapeDtypeStruct((N, D), jnp.float32),
    )(x, s4, deg, w2, b2, wu1, bu1, wu2, bu2, gamma, beta)


def kernel(node_features, edge_index, edge_features, W1, b1, W2, b2,
           Wu1, bu1, Wu2, bu2, gamma, beta):
    src = edge_index[0]
    dst = edge_index[1]
    offs = jnp.arange(2 * NG, dtype=jnp.int32) * N
    idx_all = (jnp.where(offs[:, None] < NG * N, src[None, :], dst[None, :])
               + offs[:, None]).reshape(-1)

    p8 = _tc_proj(node_features, W1)
    g8 = _sc_gather(p8.reshape(2 * NG * N, GC), idx_all)
    hm = _tc_gelu(g8.reshape(2 * NG, E, GC), edge_features,
                  W1[2 * D:], b1.reshape(1, H))
    s_flat, deg = _sc_scatter(hm.reshape(NG * E, GC), dst)
    out = _tc_update(node_features, s_flat.reshape(NG, N, GC), deg,
                     W2, b2.reshape(1, D), Wu1, bu1.reshape(1, H),
                     Wu2, bu2.reshape(1, D), gamma.reshape(1, D),
                     beta.reshape(1, D))
    return out


# speedup vs baseline: 2.3239x; 1.4793x over previous
import functools

import jax
import jax.numpy as jnp
from jax import lax
from jax.experimental import pallas as pl
from jax.experimental.pallas import tpu as pltpu
from jax.experimental.pallas import tpu_sc as plsc

N = 10000
E = 160000
D = 256
DE = 16
H = 512
GC = 128
NG = H // GC
CHUNK = 128
NCH = E // CHUNK
NWORK = 32
FLUSH_ROWS = 624
FLUSH_TAIL = N - 16 * FLUSH_ROWS
CH2 = 40
NJOB = E // CH2
JROWS = 8 * CH2


def _proj_body(x_ref, w_ref, out_ref):
    x = x_ref[...]
    for t in range(2 * NG):
        r0 = D * (t // NG)
        c0 = GC * (t % NG)
        out_ref[t] = jnp.dot(x, w_ref[r0:r0 + D, c0:c0 + GC],
                             preferred_element_type=jnp.float32)


def _tc_proj(x, w1):
    nb = 2000
    return pl.pallas_call(
        _proj_body,
        grid=(N // nb,),
        in_specs=[
            pl.BlockSpec((nb, D), lambda n: (n, 0)),
            pl.BlockSpec((2 * D + DE, H), lambda n: (0, 0)),
        ],
        out_specs=pl.BlockSpec((2 * NG, nb, GC), lambda n: (0, n, 0)),
        out_shape=jax.ShapeDtypeStruct((2 * NG, N, GC), jnp.float32),
    )(x, w1)


def _gather_body(p8, idx_h, out, idxa, idxb, bufa, bufb,
                 gsema, gsemb, wsema, wsemb):
    c = lax.axis_index("c")
    s = lax.axis_index("s")
    wid = s * 2 + c

    def load_idx(ibuf, j):
        r = wid + NWORK * j
        pltpu.sync_copy(idx_h.at[pl.ds(r * JROWS, JROWS)], ibuf)

    def issue_gathers(ibuf, dbuf, gsem):
        for t in range(2 * NG):
            pltpu.async_copy(p8.at[ibuf.at[pl.ds(t * CH2, CH2)]],
                             dbuf.at[pl.ds(t * CH2, CH2)], gsem)

    def wait_gathers(dbuf, gsem):
        pltpu.make_async_copy(p8.at[pl.ds(0, JROWS)], dbuf, gsem).wait()

    def issue_write(dbuf, j, wsem):
        r = wid + NWORK * j
        pltpu.async_copy(dbuf, out.at[pl.ds(r * JROWS, JROWS)], wsem)

    def wait_write(dbuf, wsem):
        pltpu.make_async_copy(dbuf, out.at[pl.ds(0, JROWS)], wsem).wait()

    load_idx(idxa, 0)
    issue_gathers(idxa, bufa, gsema)

    def body(k, carry):
        a = 2 * k
        b = 2 * k + 1

        @pl.when(k > 0)
        def _():
            wait_write(bufb, wsemb)

        load_idx(idxb, b)
        issue_gathers(idxb, bufb, gsemb)
        wait_gathers(bufa, gsema)
        issue_write(bufa, a, wsema)
        wait_gathers(bufb, gsemb)
        issue_write(bufb, b, wsemb)
        wait_write(bufa, wsema)
        load_idx(idxa, a + 2)
        issue_gathers(idxa, bufa, gsema)
        return carry

    lax.fori_loop(0, 62, body, 0)
    wait_gathers(bufa, gsema)
    issue_write(bufa, 124, wsema)
    wait_write(bufb, wsemb)
    wait_write(bufa, wsema)


def _sc_gather(p8_flat, idx_all):
    mesh = plsc.VectorSubcoreMesh(core_axis_name="c", subcore_axis_name="s")
    f = pl.kernel(
        _gather_body,
        out_type=jax.ShapeDtypeStruct((2 * NG * E, GC), jnp.float32),
        mesh=mesh,
        scratch_types=[
            pltpu.VMEM((JROWS,), jnp.int32),
            pltpu.VMEM((JROWS,), jnp.int32),
            pltpu.VMEM((JROWS, GC), jnp.float32),
            pltpu.VMEM((JROWS, GC), jnp.float32),
            pltpu.SemaphoreType.DMA,
            pltpu.SemaphoreType.DMA,
            pltpu.SemaphoreType.DMA,
            pltpu.SemaphoreType.DMA,
        ],
    )
    return f(p8_flat, idx_all)


_INV_SQRT2 = 0.7071067811865476


def _gelu_body(g8_ref, ef_ref, wc_ref, b1_ref, out_ref):
    be = ef_ref.shape[0]
    ef = ef_ref[...]
    for g in range(NG):
        za = g8_ref[:, g].reshape(be, GC)
        zb = g8_ref[:, NG + g].reshape(be, GC)
        z = (za + zb
             + jnp.dot(ef, wc_ref[:, g * GC:(g + 1) * GC],
                       preferred_element_type=jnp.float32)
             + b1_ref[0, g * GC:(g + 1) * GC][None, :])
        out_ref[g] = z * 0.5 * (1.0 + lax.erf(z * _INV_SQRT2))


def _tc_gelu(g8, ef, wc, b1_2d):
    nbr = 25
    be = nbr * CH2
    return pl.pallas_call(
        _gelu_body,
        grid=(NJOB // nbr,),
        in_specs=[
            pl.BlockSpec((nbr, 2 * NG, CH2, GC), lambda e: (e, 0, 0, 0)),
            pl.BlockSpec((be, DE), lambda e: (e, 0)),
            pl.BlockSpec((DE, H), lambda e: (0, 0)),
            pl.BlockSpec((1, H), lambda e: (0, 0)),
        ],
        out_specs=pl.BlockSpec((NG, be, GC), lambda e: (0, e, 0)),
        out_shape=jax.ShapeDtypeStruct((NG, E, GC), jnp.float32),
    )(g8, ef, wc, b1_2d)


def _scatter_body(hm, dst_h, s_out, deg_out, acc, hma, hmb, idxa, idxb,
                  lsema, lsemb, ssema, ssemb):
    c = lax.axis_index("c")
    s = lax.axis_index("s")
    base = s * FLUSH_ROWS
    zero_chunks = [(0, CHUNK), (CHUNK, CHUNK), (2 * CHUNK, CHUNK),
                   (3 * CHUNK, CHUNK), (4 * CHUNK, FLUSH_ROWS - 4 * CHUNK)]

    def fill_hma(val):
        def fill(j, carry):
            hma[j // 8, pl.ds((j % 8) * 16, 16)] = jnp.full((16,), val,
                                                            jnp.float32)
            return carry
        lax.fori_loop(0, CHUNK * (GC // 16), fill, 0)

    def zero_acc():
        fill_hma(0.0)
        for off, ln in zero_chunks:
            pltpu.sync_copy(hma.at[pl.ds(0, ln)],
                            acc.at[pl.ds(base + off, ln)])

        @pl.when(s == 0)
        def _():
            pltpu.sync_copy(hma.at[pl.ds(0, FLUSH_TAIL)],
                            acc.at[pl.ds(16 * FLUSH_ROWS, FLUSH_TAIL)])

    def flush_acc(out_ref, row0):
        pltpu.sync_copy(acc.at[pl.ds(base, FLUSH_ROWS)],
                        out_ref.at[pl.ds(row0 + base, FLUSH_ROWS)])

        @pl.when(s == 0)
        def _():
            pltpu.sync_copy(
                acc.at[pl.ds(16 * FLUSH_ROWS, FLUSH_TAIL)],
                out_ref.at[pl.ds(row0 + 16 * FLUSH_ROWS, FLUSH_TAIL)])

    def load_idx(ibuf, lsem, j):
        r = s + 16 * j
        pltpu.async_copy(dst_h.at[pl.ds(r * CHUNK, CHUNK)], ibuf, lsem)

    def wait_idx(ibuf, lsem):
        pltpu.make_async_copy(dst_h.at[pl.ds(0, CHUNK)], ibuf, lsem).wait()

    def load_hm(dbuf, lsem, g, j):
        r = s + 16 * j
        pltpu.async_copy(hm.at[pl.ds(g * E + r * CHUNK, CHUNK)], dbuf, lsem)

    def wait_hm(dbuf, lsem):
        pltpu.make_async_copy(hm.at[pl.ds(0, CHUNK)], dbuf, lsem).wait()

    def issue_scat(dbuf, ibuf, ssem):
        pltpu.async_copy(dbuf, acc.at[ibuf], ssem, add=True)

    def wait_scat(dbuf, ssem):
        pltpu.make_async_copy(dbuf, acc.at[pl.ds(0, CHUNK)], ssem).wait()

    for gg in range(2):
        g = c * 2 + gg
        zero_acc()
        plsc.subcore_barrier()

        load_idx(idxa, lsema, 0)
        load_hm(hma, lsema, g, 0)

        def body(k, carry):
            a = 2 * k
            b = 2 * k + 1

            @pl.when(k > 0)
            def _():
                wait_scat(hmb, ssemb)

            load_idx(idxb, lsemb, b)
            load_hm(hmb, lsemb, g, b)
            wait_idx(idxa, lsema)
            wait_hm(hma, lsema)
            issue_scat(hma, idxa, ssema)
            wait_idx(idxb, lsemb)
            wait_hm(hmb, lsemb)
            wait_scat(hma, ssema)

            @pl.when(k < 38)
            def _():
                load_idx(idxa, lsema, a + 2)
                load_hm(hma, lsema, g, a + 2)

            issue_scat(hmb, idxb, ssemb)
            return carry

        lax.fori_loop(0, 39, body, 0)
        wait_scat(hmb, ssemb)

        @pl.when(s < 2)
        def _():
            r = 16 * 78 + s
            pltpu.sync_copy(dst_h.at[pl.ds(r * CHUNK, CHUNK)], idxa)
            pltpu.sync_copy(hm.at[pl.ds(g * E + r * CHUNK, CHUNK)], hma)
            pltpu.sync_copy(hma, acc.at[idxa], add=True)

        plsc.subcore_barrier()
        flush_acc(s_out, g * N)
        plsc.subcore_barrier()

    @pl.when(c == 0)
    def _():
        zero_acc()
    plsc.subcore_barrier()

    @pl.when(c == 0)
    def _():
        fill_hma(1.0)
        load_idx(idxa, lsema, 0)

        def dbody(k, carry):
            a = 2 * k
            b = 2 * k + 1

            @pl.when(k > 0)
            def _():
                wait_scat(hma, ssemb)

            load_idx(idxb, lsemb, b)
            wait_idx(idxa, lsema)
            issue_scat(hma, idxa, ssema)
            wait_idx(idxb, lsemb)
            wait_scat(hma, ssema)

            @pl.when(k < 38)
            def _():
                load_idx(idxa, lsema, a + 2)

            issue_scat(hma, idxb, ssemb)
            return carry

        lax.fori_loop(0, 39, dbody, 0)
        wait_scat(hma, ssemb)

        @pl.when(s < 2)
        def _():
            r = 16 * 78 + s
            pltpu.sync_copy(dst_h.at[pl.ds(r * CHUNK, CHUNK)], idxa)
            pltpu.sync_copy(hma, acc.at[idxa], add=True)

    plsc.subcore_barrier()

    @pl.when(c == 0)
    def _():
        flush_acc(deg_out, 0)


def _sc_scatter(hm_flat, dst):
    mesh = plsc.VectorSubcoreMesh(core_axis_name="c", subcore_axis_name="s")
    f = pl.kernel(
        _scatter_body,
        out_type=[
            jax.ShapeDtypeStruct((NG * N, GC), jnp.float32),
            jax.ShapeDtypeStruct((N, GC), jnp.float32),
        ],
        mesh=mesh,
        scratch_types=[
            pltpu.VMEM_SHARED((N, GC), jnp.float32),
            pltpu.VMEM((CHUNK, GC), jnp.float32),
            pltpu.VMEM((CHUNK, GC), jnp.float32),
            pltpu.VMEM((CHUNK,), jnp.int32),
            pltpu.VMEM((CHUNK,), jnp.int32),
            pltpu.SemaphoreType.DMA,
            pltpu.SemaphoreType.DMA,
            pltpu.SemaphoreType.DMA,
            pltpu.SemaphoreType.DMA,
        ],
    )
    return f(hm_flat, dst)


def _update_body(x_ref, s_ref, deg_ref, w2_ref, b2_ref, wu1_ref, bu1_ref,
                 wu2_ref, bu2_ref, gamma_ref, beta_ref, out_ref):
    x = x_ref[...]
    sw = jnp.dot(s_ref[0], w2_ref[0:GC, :], preferred_element_type=jnp.float32)
    for g in range(1, NG):
        sw = sw + jnp.dot(s_ref[g], w2_ref[g * GC:(g + 1) * GC, :],
                          preferred_element_type=jnp.float32)
    deg = deg_ref[:, 0:1]
    md = jnp.maximum(deg, 1.0)
    agg = sw / md + (deg / md) * b2_ref[...]
    t = (jnp.dot(x, wu1_ref[0:D, :], preferred_element_type=jnp.float32)
         + jnp.dot(agg, wu1_ref[D:2 * D, :], preferred_element_type=jnp.float32)
         + bu1_ref[...])
    u = t * 0.5 * (1.0 + lax.erf(t * _INV_SQRT2))
    y = (jnp.dot(u, wu2_ref[...], preferred_element_type=jnp.float32)
         + bu2_ref[...] + x)
    mu = jnp.mean(y, axis=-1, keepdims=True)
    d = y - mu
    var = jnp.mean(d * d, axis=-1, keepdims=True)
    out_ref[...] = d * lax.rsqrt(var + 1e-5) * gamma_ref[...] + beta_ref[...]


def _tc_update(x, s4, deg, w2, b2, wu1, bu1, wu2, bu2, gamma, beta):
    nb = 1000
    full = lambda shape: pl.BlockSpec(shape, lambda n: tuple(0 for _ in shape))
    return pl.pallas_call(
        _update_body,
        grid=(N // nb,),
        in_specs=[
            pl.BlockSpec((nb, D), lambda n: (n, 0)),
            pl.BlockSpec((NG, nb, GC), lambda n: (0, n, 0)),
            pl.BlockSpec((nb, GC), lambda n: (n, 0)),
            full((H, D)),
            full((1, D)),
            full((2 * D, H)),
            full((1, H)),
            full((H, D)),
            full((1, D)),
            full((1, D)),
            full((1, D)),
        ],
        out_specs=pl.BlockSpec((nb, D), lambda n: (n, 0)),
        out_shape=jax.ShapeDtypeStruct((N, D), jnp.float32),
    )(x, s4, deg, w2, b2, wu1, bu1, wu2, bu2, gamma, beta)


def kernel(node_features, edge_index, edge_features, W1, b1, W2, b2,
           Wu1, bu1, Wu2, bu2, gamma, beta):
    src = edge_index[0]
    dst = edge_index[1]
    offs = jnp.arange(2 * NG, dtype=jnp.int32) * N
    idx8 = (jnp.where(offs[:, None] < NG * N, src[None, :], dst[None, :])
            + offs[:, None])
    idx_all = jnp.transpose(idx8.reshape(2 * NG, NJOB, CH2),
                            (1, 0, 2)).reshape(-1)

    p8 = _tc_proj(node_features, W1)
    g8 = _sc_gather(p8.reshape(2 * NG * N, GC), idx_all)
    hm = _tc_gelu(g8.reshape(NJOB, 2 * NG, CH2, GC), edge_features,
                  W1[2 * D:], b1.reshape(1, H))
    s_flat, deg = _sc_scatter(hm.reshape(NG * E, GC), dst)
    out = _tc_update(node_features, s_flat.reshape(NG, N, GC), deg,
                     W2, b2.reshape(1, D), Wu1, bu1.reshape(1, H),
                     Wu2, bu2.reshape(1, D), gamma.reshape(1, D),
                     beta.reshape(1, D))
    return out


# speedup vs baseline: 2.5406x; 1.0932x over previous
import functools

import jax
import jax.numpy as jnp
from jax import lax
from jax.experimental import pallas as pl
from jax.experimental.pallas import tpu as pltpu
from jax.experimental.pallas import tpu_sc as plsc

N = 10000
E = 160000
D = 256
DE = 16
H = 512
GC = 128
NG = H // GC
CHUNK = 128
NCH = E // CHUNK
NWORK = 32
FLUSH_ROWS = 624
FLUSH_TAIL = N - 16 * FLUSH_ROWS
CH2 = 40
NJOB = E // CH2
JROWS = 8 * CH2
OROWS = NG * CH2


def _proj_body(x_ref, w_ref, out_ref):
    x = x_ref[...]
    for t in range(2 * NG):
        r0 = D * (t // NG)
        c0 = GC * (t % NG)
        out_ref[t] = jnp.dot(x, w_ref[r0:r0 + D, c0:c0 + GC],
                             preferred_element_type=jnp.float32)


def _tc_proj(x, w1):
    nb = 2000
    return pl.pallas_call(
        _proj_body,
        grid=(N // nb,),
        in_specs=[
            pl.BlockSpec((nb, D), lambda n: (n, 0)),
            pl.BlockSpec((2 * D + DE, H), lambda n: (0, 0)),
        ],
        out_specs=pl.BlockSpec((2 * NG, nb, GC), lambda n: (0, n, 0)),
        out_shape=jax.ShapeDtypeStruct((2 * NG, N, GC), jnp.float32),
    )(x, w1)


def _gather_body(p8, idx_h, out, idxa, idxb, bufa, bufb,
                 gsema, gsemb, wsema, wsemb):
    c = lax.axis_index("c")
    s = lax.axis_index("s")
    wid = s * 2 + c

    def load_idx(ibuf, j):
        r = wid + NWORK * j
        pltpu.sync_copy(idx_h.at[pl.ds(r * JROWS, JROWS)], ibuf)

    def issue_gathers(ibuf, dbuf, gsem):
        for t in range(2 * NG):
            pltpu.async_copy(p8.at[ibuf.at[pl.ds(t * CH2, CH2)]],
                             dbuf.at[pl.ds(t * CH2, CH2)], gsem)

    def wait_gathers(dbuf, gsem):
        pltpu.make_async_copy(p8.at[pl.ds(0, JROWS)], dbuf, gsem).wait()

    def pair_sum(dbuf):
        def addrow(j, carry):
            for g in range(NG):
                for k in range(GC // 16):
                    x = dbuf[(NG + g) * CH2 + j, pl.ds(k * 16, 16)]
                    plsc.addupdate(dbuf.at[g * CH2 + j, pl.ds(k * 16, 16)], x)
            return carry
        lax.fori_loop(0, CH2, addrow, 0)

    def issue_write(dbuf, j, wsem):
        r = wid + NWORK * j
        pltpu.async_copy(dbuf.at[pl.ds(0, OROWS)],
                         out.at[pl.ds(r * OROWS, OROWS)], wsem)

    def wait_write(dbuf, wsem):
        pltpu.make_async_copy(dbuf.at[pl.ds(0, OROWS)],
                              out.at[pl.ds(0, OROWS)], wsem).wait()

    load_idx(idxa, 0)
    issue_gathers(idxa, bufa, gsema)

    def body(k, carry):
        a = 2 * k
        b = 2 * k + 1

        @pl.when(k > 0)
        def _():
            wait_write(bufb, wsemb)

        load_idx(idxb, b)
        issue_gathers(idxb, bufb, gsemb)
        wait_gathers(bufa, gsema)
        pair_sum(bufa)
        issue_write(bufa, a, wsema)
        wait_gathers(bufb, gsemb)
        pair_sum(bufb)
        issue_write(bufb, b, wsemb)
        wait_write(bufa, wsema)
        load_idx(idxa, a + 2)
        issue_gathers(idxa, bufa, gsema)
        return carry

    lax.fori_loop(0, 62, body, 0)
    wait_gathers(bufa, gsema)
    pair_sum(bufa)
    issue_write(bufa, 124, wsema)
    wait_write(bufb, wsemb)
    wait_write(bufa, wsema)


def _sc_gather(p8_flat, idx_all):
    mesh = plsc.VectorSubcoreMesh(core_axis_name="c", subcore_axis_name="s")
    f = pl.kernel(
        _gather_body,
        out_type=jax.ShapeDtypeStruct((NG * E, GC), jnp.float32),
        mesh=mesh,
        scratch_types=[
            pltpu.VMEM((JROWS,), jnp.int32),
            pltpu.VMEM((JROWS,), jnp.int32),
            pltpu.VMEM((JROWS, GC), jnp.float32),
            pltpu.VMEM((JROWS, GC), jnp.float32),
            pltpu.SemaphoreType.DMA,
            pltpu.SemaphoreType.DMA,
            pltpu.SemaphoreType.DMA,
            pltpu.SemaphoreType.DMA,
        ],
    )
    return f(p8_flat, idx_all)


_INV_SQRT2 = 0.7071067811865476


def _gelu_body(g8_ref, ef_ref, wc_ref, b1_ref, out_ref):
    be = ef_ref.shape[0]
    ef = ef_ref[...]
    for g in range(NG):
        z = (g8_ref[:, g].reshape(be, GC)
             + jnp.dot(ef, wc_ref[:, g * GC:(g + 1) * GC],
                       preferred_element_type=jnp.float32)
             + b1_ref[0, g * GC:(g + 1) * GC][None, :])
        out_ref[g] = z * 0.5 * (1.0 + lax.erf(z * _INV_SQRT2))


def _tc_gelu(g8, ef, wc, b1_2d):
    nbr = 25
    be = nbr * CH2
    return pl.pallas_call(
        _gelu_body,
        grid=(NJOB // nbr,),
        in_specs=[
            pl.BlockSpec((nbr, NG, CH2, GC), lambda e: (e, 0, 0, 0)),
            pl.BlockSpec((be, DE), lambda e: (e, 0)),
            pl.BlockSpec((DE, H), lambda e: (0, 0)),
            pl.BlockSpec((1, H), lambda e: (0, 0)),
        ],
        out_specs=pl.BlockSpec((NG, be, GC), lambda e: (0, e, 0)),
        out_shape=jax.ShapeDtypeStruct((NG, E, GC), jnp.float32),
    )(g8, ef, wc, b1_2d)


def _scatter_body(hm, dst_h, s_out, deg_out, acc, hma, hmb, idxa, idxb,
                  lsema, lsemb, ssema, ssemb):
    c = lax.axis_index("c")
    s = lax.axis_index("s")
    base = s * FLUSH_ROWS
    zero_chunks = [(0, CHUNK), (CHUNK, CHUNK), (2 * CHUNK, CHUNK),
                   (3 * CHUNK, CHUNK), (4 * CHUNK, FLUSH_ROWS - 4 * CHUNK)]

    def fill_hma(val):
        def fill(j, carry):
            hma[j // 8, pl.ds((j % 8) * 16, 16)] = jnp.full((16,), val,
                                                            jnp.float32)
            return carry
        lax.fori_loop(0, CHUNK * (GC // 16), fill, 0)

    def zero_acc():
        fill_hma(0.0)
        for off, ln in zero_chunks:
            pltpu.sync_copy(hma.at[pl.ds(0, ln)],
                            acc.at[pl.ds(base + off, ln)])

        @pl.when(s == 0)
        def _():
            pltpu.sync_copy(hma.at[pl.ds(0, FLUSH_TAIL)],
                            acc.at[pl.ds(16 * FLUSH_ROWS, FLUSH_TAIL)])

    def flush_acc(out_ref, row0):
        pltpu.sync_copy(acc.at[pl.ds(base, FLUSH_ROWS)],
                        out_ref.at[pl.ds(row0 + base, FLUSH_ROWS)])

        @pl.when(s == 0)
        def _():
            pltpu.sync_copy(
                acc.at[pl.ds(16 * FLUSH_ROWS, FLUSH_TAIL)],
                out_ref.at[pl.ds(row0 + 16 * FLUSH_ROWS, FLUSH_TAIL)])

    def load_idx(ibuf, lsem, j):
        r = s + 16 * j
        pltpu.async_copy(dst_h.at[pl.ds(r * CHUNK, CHUNK)], ibuf, lsem)

    def wait_idx(ibuf, lsem):
        pltpu.make_async_copy(dst_h.at[pl.ds(0, CHUNK)], ibuf, lsem).wait()

    def load_hm(dbuf, lsem, g, j):
        r = s + 16 * j
        pltpu.async_copy(hm.at[pl.ds(g * E + r * CHUNK, CHUNK)], dbuf, lsem)

    def wait_hm(dbuf, lsem):
        pltpu.make_async_copy(hm.at[pl.ds(0, CHUNK)], dbuf, lsem).wait()

    def issue_scat(dbuf, ibuf, ssem):
        pltpu.async_copy(dbuf, acc.at[ibuf], ssem, add=True)

    def wait_scat(dbuf, ssem):
        pltpu.make_async_copy(dbuf, acc.at[pl.ds(0, CHUNK)], ssem).wait()

    for gg in range(2):
        g = c * 2 + gg
        zero_acc()
        plsc.subcore_barrier()

        load_idx(idxa, lsema, 0)
        load_hm(hma, lsema, g, 0)

        def body(k, carry):
            a = 2 * k
            b = 2 * k + 1

            @pl.when(k > 0)
            def _():
                wait_scat(hmb, ssemb)

            load_idx(idxb, lsemb, b)
            load_hm(hmb, lsemb, g, b)
            wait_idx(idxa, lsema)
            wait_hm(hma, lsema)
            issue_scat(hma, idxa, ssema)
            wait_idx(idxb, lsemb)
            wait_hm(hmb, lsemb)
            wait_scat(hma, ssema)

            @pl.when(k < 38)
            def _():
                load_idx(idxa, lsema, a + 2)
                load_hm(hma, lsema, g, a + 2)

            issue_scat(hmb, idxb, ssemb)
            return carry

        lax.fori_loop(0, 39, body, 0)
        wait_scat(hmb, ssemb)

        @pl.when(s < 2)
        def _():
            r = 16 * 78 + s
            pltpu.sync_copy(dst_h.at[pl.ds(r * CHUNK, CHUNK)], idxa)
            pltpu.sync_copy(hm.at[pl.ds(g * E + r * CHUNK, CHUNK)], hma)
            pltpu.sync_copy(hma, acc.at[idxa], add=True)

        plsc.subcore_barrier()
        flush_acc(s_out, g * N)
        plsc.subcore_barrier()

    @pl.when(c == 1)
    def _():
        zero_acc()
    plsc.subcore_barrier()

    @pl.when(c == 1)
    def _():
        fill_hma(1.0)
        load_idx(idxa, lsema, 0)

        def dbody(k, carry):
            a = 2 * k
            b = 2 * k + 1

            @pl.when(k > 0)
            def _():
                wait_scat(hma, ssemb)

            load_idx(idxb, lsemb, b)
            wait_idx(idxa, lsema)
            issue_scat(hma, idxa, ssema)
            wait_idx(idxb, lsemb)
            wait_scat(hma, ssema)

            @pl.when(k < 38)
            def _():
                load_idx(idxa, lsema, a + 2)

            issue_scat(hma, idxb, ssemb)
            return carry

        lax.fori_loop(0, 39, dbody, 0)
        wait_scat(hma, ssemb)

        @pl.when(s < 2)
        def _():
            r = 16 * 78 + s
            pltpu.sync_copy(dst_h.at[pl.ds(r * CHUNK, CHUNK)], idxa)
            pltpu.sync_copy(hma, acc.at[idxa], add=True)

    plsc.subcore_barrier()

    @pl.when(c == 1)
    def _():
        flush_acc(deg_out, 0)


def _sc_scatter(hm_flat, dst):
    mesh = plsc.VectorSubcoreMesh(core_axis_name="c", subcore_axis_name="s")
    f = pl.kernel(
        _scatter_body,
        out_type=[
            jax.ShapeDtypeStruct((NG * N, GC), jnp.float32),
            jax.ShapeDtypeStruct((N, GC), jnp.float32),
        ],
        mesh=mesh,
        scratch_types=[
            pltpu.VMEM_SHARED((N, GC), jnp.float32),
            pltpu.VMEM((CHUNK, GC), jnp.float32),
            pltpu.VMEM((CHUNK, GC), jnp.float32),
            pltpu.VMEM((CHUNK,), jnp.int32),
            pltpu.VMEM((CHUNK,), jnp.int32),
            pltpu.SemaphoreType.DMA,
            pltpu.SemaphoreType.DMA,
            pltpu.SemaphoreType.DMA,
            pltpu.SemaphoreType.DMA,
        ],
    )
    return f(hm_flat, dst)


def _update_body(x_ref, s_ref, deg_ref, w2_ref, b2_ref, wu1_ref, bu1_ref,
                 wu2_ref, bu2_ref, gamma_ref, beta_ref, out_ref):
    x = x_ref[...]
    sw = jnp.dot(s_ref[0], w2_ref[0:GC, :], preferred_element_type=jnp.float32)
    for g in range(1, NG):
        sw = sw + jnp.dot(s_ref[g], w2_ref[g * GC:(g + 1) * GC, :],
                          preferred_element_type=jnp.float32)
    deg = deg_ref[:, 0:1]
    md = jnp.maximum(deg, 1.0)
    agg = sw / md + (deg / md) * b2_ref[...]
    t = (jnp.dot(x, wu1_ref[0:D, :], preferred_element_type=jnp.float32)
         + jnp.dot(agg, wu1_ref[D:2 * D, :], preferred_element_type=jnp.float32)
         + bu1_ref[...])
    u = t * 0.5 * (1.0 + lax.erf(t * _INV_SQRT2))
    y = (jnp.dot(u, wu2_ref[...], preferred_element_type=jnp.float32)
         + bu2_ref[...] + x)
    mu = jnp.mean(y, axis=-1, keepdims=True)
    d = y - mu
    var = jnp.mean(d * d, axis=-1, keepdims=True)
    out_ref[...] = d * lax.rsqrt(var + 1e-5) * gamma_ref[...] + beta_ref[...]


def _tc_update(x, s4, deg, w2, b2, wu1, bu1, wu2, bu2, gamma, beta):
    nb = 1000
    full = lambda shape: pl.BlockSpec(shape, lambda n: tuple(0 for _ in shape))
    return pl.pallas_call(
        _update_body,
        grid=(N // nb,),
        in_specs=[
            pl.BlockSpec((nb, D), lambda n: (n, 0)),
            pl.BlockSpec((NG, nb, GC), lambda n: (0, n, 0)),
            pl.BlockSpec((nb, GC), lambda n: (n, 0)),
            full((H, D)),
            full((1, D)),
            full((2 * D, H)),
            full((1, H)),
            full((H, D)),
            full((1, D)),
            full((1, D)),
            full((1, D)),
        ],
        out_specs=pl.BlockSpec((nb, D), lambda n: (n, 0)),
        out_shape=jax.ShapeDtypeStruct((N, D), jnp.float32),
    )(x, s4, deg, w2, b2, wu1, bu1, wu2, bu2, gamma, beta)


def kernel(node_features, edge_index, edge_features, W1, b1, W2, b2,
           Wu1, bu1, Wu2, bu2, gamma, beta):
    src = edge_index[0]
    dst = edge_index[1]
    offs = jnp.arange(2 * NG, dtype=jnp.int32) * N
    idx8 = (jnp.where(offs[:, None] < NG * N, src[None, :], dst[None, :])
            + offs[:, None])
    idx_all = jnp.transpose(idx8.reshape(2 * NG, NJOB, CH2),
                            (1, 0, 2)).reshape(-1)

    p8 = _tc_proj(node_features, W1)
    g8 = _sc_gather(p8.reshape(2 * NG * N, GC), idx_all)
    hm = _tc_gelu(g8.reshape(NJOB, NG, CH2, GC), edge_features,
                  W1[2 * D:], b1.reshape(1, H))
    s_flat, deg = _sc_scatter(hm.reshape(NG * E, GC), dst)
    out = _tc_update(node_features, s_flat.reshape(NG, N, GC), deg,
                     W2, b2.reshape(1, D), Wu1, bu1.reshape(1, H),
                     Wu2, bu2.reshape(1, D), gamma.reshape(1, D),
                     beta.reshape(1, D))
    return out


# speedup vs baseline: 2.7027x; 1.0638x over previous
import functools

import jax
import jax.numpy as jnp
from jax import lax
from jax.experimental import pallas as pl
from jax.experimental.pallas import tpu as pltpu
from jax.experimental.pallas import tpu_sc as plsc

N = 10000
E = 160000
D = 256
DE = 16
H = 512
GC = 128
NG = H // GC
NGH = 2
CHUNK = 128
NCH = E // CHUNK
NWORK = 32
FLUSH_ROWS = 624
FLUSH_TAIL = N - 16 * FLUSH_ROWS
CH2 = 40
NJOB = E // CH2
JROWS = 2 * NGH * CH2
OROWS = NGH * CH2


def _proj_body(x_ref, w_ref, out_ref):
    x = x_ref[...]
    for t in range(2 * NG):
        r0 = D * (t // NG)
        c0 = GC * (t % NG)
        out_ref[t] = jnp.dot(x, w_ref[r0:r0 + D, c0:c0 + GC],
                             preferred_element_type=jnp.float32)


def _tc_proj(x, w1):
    nb = 2000
    return pl.pallas_call(
        _proj_body,
        grid=(N // nb,),
        in_specs=[
            pl.BlockSpec((nb, D), lambda n: (n, 0)),
            pl.BlockSpec((2 * D + DE, H), lambda n: (0, 0)),
        ],
        out_specs=pl.BlockSpec((2 * NG, nb, GC), lambda n: (0, n, 0)),
        out_shape=jax.ShapeDtypeStruct((2 * NG, N, GC), jnp.float32),
    )(x, w1)


def _gather_body(p8, idx_h, out, idxa, idxb, bufa, bufb,
                 gsema, gsemb, wsema, wsemb):
    c = lax.axis_index("c")
    s = lax.axis_index("s")
    wid = s * 2 + c

    def load_idx(ibuf, j):
        r = wid + NWORK * j
        pltpu.sync_copy(idx_h.at[pl.ds(r * JROWS, JROWS)], ibuf)

    def issue_gathers(ibuf, dbuf, gsem):
        for t in range(2 * NGH):
            pltpu.async_copy(p8.at[ibuf.at[pl.ds(t * CH2, CH2)]],
                             dbuf.at[pl.ds(t * CH2, CH2)], gsem)

    def wait_gathers(dbuf, gsem):
        pltpu.make_async_copy(p8.at[pl.ds(0, JROWS)], dbuf, gsem).wait()

    def pair_sum(dbuf):
        def addrow(j, carry):
            for g in range(NGH):
                for k in range(GC // 16):
                    x = dbuf[(NGH + g) * CH2 + j, pl.ds(k * 16, 16)]
                    plsc.addupdate(dbuf.at[g * CH2 + j, pl.ds(k * 16, 16)], x)
            return carry
        lax.fori_loop(0, CH2, addrow, 0)

    def issue_write(dbuf, j, wsem):
        r = wid + NWORK * j
        pltpu.async_copy(dbuf.at[pl.ds(0, OROWS)],
                         out.at[pl.ds(r * OROWS, OROWS)], wsem)

    def wait_write(dbuf, wsem):
        pltpu.make_async_copy(dbuf.at[pl.ds(0, OROWS)],
                              out.at[pl.ds(0, OROWS)], wsem).wait()

    load_idx(idxa, 0)
    issue_gathers(idxa, bufa, gsema)

    def body(k, carry):
        a = 2 * k
        b = 2 * k + 1

        @pl.when(k > 0)
        def _():
            wait_write(bufb, wsemb)

        load_idx(idxb, b)
        issue_gathers(idxb, bufb, gsemb)
        wait_gathers(bufa, gsema)
        pair_sum(bufa)
        issue_write(bufa, a, wsema)
        wait_gathers(bufb, gsemb)
        pair_sum(bufb)
        issue_write(bufb, b, wsemb)
        wait_write(bufa, wsema)
        load_idx(idxa, a + 2)
        issue_gathers(idxa, bufa, gsema)
        return carry

    lax.fori_loop(0, 62, body, 0)
    wait_gathers(bufa, gsema)
    pair_sum(bufa)
    issue_write(bufa, 124, wsema)
    wait_write(bufb, wsemb)
    wait_write(bufa, wsema)


def _sc_gather(p8_flat, idx_half):
    mesh = plsc.VectorSubcoreMesh(core_axis_name="c", subcore_axis_name="s")
    f = pl.kernel(
        _gather_body,
        out_type=jax.ShapeDtypeStruct((NGH * E, GC), jnp.float32),
        mesh=mesh,
        scratch_types=[
            pltpu.VMEM((JROWS,), jnp.int32),
            pltpu.VMEM((JROWS,), jnp.int32),
            pltpu.VMEM((JROWS, GC), jnp.float32),
            pltpu.VMEM((JROWS, GC), jnp.float32),
            pltpu.SemaphoreType.DMA,
            pltpu.SemaphoreType.DMA,
            pltpu.SemaphoreType.DMA,
            pltpu.SemaphoreType.DMA,
        ],
    )
    return f(p8_flat, idx_half)


_INV_SQRT2 = 0.7071067811865476


def _gelu_body(g8_ref, ef_ref, wc_ref, b1_ref, out_ref):
    be = ef_ref.shape[0]
    ef = ef_ref[...]
    for g in range(NGH):
        z = (g8_ref[:, g].reshape(be, GC)
             + jnp.dot(ef, wc_ref[:, g * GC:(g + 1) * GC],
                       preferred_element_type=jnp.float32)
             + b1_ref[0, g * GC:(g + 1) * GC][None, :])
        out_ref[g] = z * 0.5 * (1.0 + lax.erf(z * _INV_SQRT2))


def _tc_gelu(g8, ef, wc_h, b1_h):
    nbr = 25
    be = nbr * CH2
    return pl.pallas_call(
        _gelu_body,
        grid=(NJOB // nbr,),
        in_specs=[
            pl.BlockSpec((nbr, NGH, CH2, GC), lambda e: (e, 0, 0, 0)),
            pl.BlockSpec((be, DE), lambda e: (e, 0)),
            pl.BlockSpec((DE, NGH * GC), lambda e: (0, 0)),
            pl.BlockSpec((1, NGH * GC), lambda e: (0, 0)),
        ],
        out_specs=pl.BlockSpec((NGH, be, GC), lambda e: (0, e, 0)),
        out_shape=jax.ShapeDtypeStruct((NGH, E, GC), jnp.float32),
    )(g8, ef, wc_h, b1_h)


def _scatter_body(with_deg, hm, dst_h, *refs):
    if with_deg:
        (s_out, deg_out, acc, hma, hmb, idxa, idxb,
         lsema, lsemb, ssema, ssemb) = refs
    else:
        (s_out, acc, hma, hmb, idxa, idxb,
         lsema, lsemb, ssema, ssemb) = refs
    c = lax.axis_index("c")
    s = lax.axis_index("s")
    base = s * FLUSH_ROWS
    zero_chunks = [(0, CHUNK), (CHUNK, CHUNK), (2 * CHUNK, CHUNK),
                   (3 * CHUNK, CHUNK), (4 * CHUNK, FLUSH_ROWS - 4 * CHUNK)]

    def fill_hma(val):
        def fill(j, carry):
            hma[j // 8, pl.ds((j % 8) * 16, 16)] = jnp.full((16,), val,
                                                            jnp.float32)
            return carry
        lax.fori_loop(0, CHUNK * (GC // 16), fill, 0)

    def zero_acc():
        fill_hma(0.0)
        for off, ln in zero_chunks:
            pltpu.sync_copy(hma.at[pl.ds(0, ln)],
                            acc.at[pl.ds(base + off, ln)])

        @pl.when(s == 0)
        def _():
            pltpu.sync_copy(hma.at[pl.ds(0, FLUSH_TAIL)],
                            acc.at[pl.ds(16 * FLUSH_ROWS, FLUSH_TAIL)])

    def flush_acc(out_ref):
        pltpu.sync_copy(acc.at[pl.ds(base, FLUSH_ROWS)],
                        out_ref.at[pl.ds(c * N + base, FLUSH_ROWS)])

        @pl.when(s == 0)
        def _():
            pltpu.sync_copy(
                acc.at[pl.ds(16 * FLUSH_ROWS, FLUSH_TAIL)],
                out_ref.at[pl.ds(c * N + 16 * FLUSH_ROWS, FLUSH_TAIL)])

    def load_idx(ibuf, lsem, r):
        pltpu.async_copy(dst_h.at[pl.ds(r * CHUNK, CHUNK)], ibuf, lsem)

    def wait_idx(ibuf, lsem):
        pltpu.make_async_copy(dst_h.at[pl.ds(0, CHUNK)], ibuf, lsem).wait()

    def load_hm(dbuf, lsem, r):
        pltpu.async_copy(hm.at[pl.ds(c * E + r * CHUNK, CHUNK)], dbuf, lsem)

    def wait_hm(dbuf, lsem):
        pltpu.make_async_copy(hm.at[pl.ds(0, CHUNK)], dbuf, lsem).wait()

    def issue_scat(dbuf, ibuf, ssem):
        pltpu.async_copy(dbuf, acc.at[ibuf], ssem, add=True)

    def wait_scat(dbuf, ssem):
        pltpu.make_async_copy(dbuf, acc.at[pl.ds(0, CHUNK)], ssem).wait()

    zero_acc()
    plsc.subcore_barrier()

    load_idx(idxa, lsema, s)
    load_hm(hma, lsema, s)

    def body(k, carry):
        ra = s + 16 * (2 * k)
        rb = s + 16 * (2 * k + 1)

        @pl.when(k > 0)
        def _():
            wait_scat(hmb, ssemb)

        load_idx(idxb, lsemb, rb)
        load_hm(hmb, lsemb, rb)
        wait_idx(idxa, lsema)
        wait_hm(hma, lsema)
        issue_scat(hma, idxa, ssema)
        wait_idx(idxb, lsemb)
        wait_hm(hmb, lsemb)
        wait_scat(hma, ssema)

        @pl.when(k < 38)
        def _():
            load_idx(idxa, lsema, ra + 32)
            load_hm(hma, lsema, ra + 32)

        issue_scat(hmb, idxb, ssemb)
        return carry

    lax.fori_loop(0, 39, body, 0)
    wait_scat(hmb, ssemb)

    @pl.when(s < 2)
    def _():
        r = 16 * 78 + s
        pltpu.sync_copy(dst_h.at[pl.ds(r * CHUNK, CHUNK)], idxa)
        pltpu.sync_copy(hm.at[pl.ds(c * E + r * CHUNK, CHUNK)], hma)
        pltpu.sync_copy(hma, acc.at[idxa], add=True)

    plsc.subcore_barrier()
    flush_acc(s_out)

    if not with_deg:
        return

    plsc.subcore_barrier()
    zero_acc()
    plsc.subcore_barrier()
    fill_hma(1.0)
    half0 = NCH // 2

    def dbody(j, carry):
        r = c * half0 + s + 16 * j
        pltpu.sync_copy(dst_h.at[pl.ds(r * CHUNK, CHUNK)], idxa)
        pltpu.sync_copy(hma, acc.at[idxa], add=True)
        return carry

    lax.fori_loop(0, (half0 - 1) // 16, dbody, 0)

    @pl.when(s == 0)
    def _():
        r = c * half0 + half0 - 1
        pltpu.sync_copy(dst_h.at[pl.ds(r * CHUNK, CHUNK)], idxa)
        pltpu.sync_copy(hma, acc.at[idxa], add=True)

    plsc.subcore_barrier()
    flush_acc(deg_out)


def _sc_scatter(hm_flat, dst, with_deg):
    mesh = plsc.VectorSubcoreMesh(core_axis_name="c", subcore_axis_name="s")
    out_type = [jax.ShapeDtypeStruct((NGH * N, GC), jnp.float32)]
    if with_deg:
        out_type.append(jax.ShapeDtypeStruct((2 * N, GC), jnp.float32))
    f = pl.kernel(
        functools.partial(_scatter_body, with_deg),
        out_type=out_type,
        mesh=mesh,
        scratch_types=[
            pltpu.VMEM_SHARED((N, GC), jnp.float32),
            pltpu.VMEM((CHUNK, GC), jnp.float32),
            pltpu.VMEM((CHUNK, GC), jnp.float32),
            pltpu.VMEM((CHUNK,), jnp.int32),
            pltpu.VMEM((CHUNK,), jnp.int32),
            pltpu.SemaphoreType.DMA,
            pltpu.SemaphoreType.DMA,
            pltpu.SemaphoreType.DMA,
            pltpu.SemaphoreType.DMA,
        ],
    )
    return f(hm_flat, dst)


def _update_body(x_ref, s0_ref, s1_ref, deg_ref, w2_ref, b2_ref, wu1_ref,
                 bu1_ref, wu2_ref, bu2_ref, gamma_ref, beta_ref, out_ref):
    x = x_ref[...]
    sw = jnp.dot(s0_ref[0], w2_ref[0:GC, :], preferred_element_type=jnp.float32)
    sw = sw + jnp.dot(s0_ref[1], w2_ref[GC:2 * GC, :],
                      preferred_element_type=jnp.float32)
    sw = sw + jnp.dot(s1_ref[0], w2_ref[2 * GC:3 * GC, :],
                      preferred_element_type=jnp.float32)
    sw = sw + jnp.dot(s1_ref[1], w2_ref[3 * GC:4 * GC, :],
                      preferred_element_type=jnp.float32)
    deg = deg_ref[0, :, 0:1] + deg_ref[1, :, 0:1]
    md = jnp.maximum(deg, 1.0)
    agg = sw / md + (deg / md) * b2_ref[...]
    t = (jnp.dot(x, wu1_ref[0:D, :], preferred_element_type=jnp.float32)
         + jnp.dot(agg, wu1_ref[D:2 * D, :], preferred_element_type=jnp.float32)
         + bu1_ref[...])
    u = t * 0.5 * (1.0 + lax.erf(t * _INV_SQRT2))
    y = (jnp.dot(u, wu2_ref[...], preferred_element_type=jnp.float32)
         + bu2_ref[...] + x)
    mu = jnp.mean(y, axis=-1, keepdims=True)
    d = y - mu
    var = jnp.mean(d * d, axis=-1, keepdims=True)
    out_ref[...] = d * lax.rsqrt(var + 1e-5) * gamma_ref[...] + beta_ref[...]


def _tc_update(x, s0, s1, deg, w2, b2, wu1, bu1, wu2, bu2, gamma, beta):
    nb = 1000
    full = lambda shape: pl.BlockSpec(shape, lambda n: tuple(0 for _ in shape))
    return pl.pallas_call(
        _update_body,
        grid=(N // nb,),
        in_specs=[
            pl.BlockSpec((nb, D), lambda n: (n, 0)),
            pl.BlockSpec((NGH, nb, GC), lambda n: (0, n, 0)),
            pl.BlockSpec((NGH, nb, GC), lambda n: (0, n, 0)),
            pl.BlockSpec((2, nb, GC), lambda n: (0, n, 0)),
            full((H, D)),
            full((1, D)),
            full((2 * D, H)),
            full((1, H)),
            full((H, D)),
            full((1, D)),
            full((1, D)),
            full((1, D)),
        ],
        out_specs=pl.BlockSpec((nb, D), lambda n: (n, 0)),
        out_shape=jax.ShapeDtypeStruct((N, D), jnp.float32),
    )(x, s0, s1, deg, w2, b2, wu1, bu1, wu2, bu2, gamma, beta)


def kernel(node_features, edge_index, edge_features, W1, b1, W2, b2,
           Wu1, bu1, Wu2, bu2, gamma, beta):
    src = edge_index[0]
    dst = edge_index[1]
    offs = jnp.arange(2 * NG, dtype=jnp.int32) * N
    idx8 = (jnp.where(offs[:, None] < NG * N, src[None, :], dst[None, :])
            + offs[:, None])

    def half_idx(h):
        tabs = jnp.array([2 * h, 2 * h + 1, NG + 2 * h, NG + 2 * h + 1])
        ih = idx8[tabs]
        return jnp.transpose(ih.reshape(2 * NGH, NJOB, CH2),
                             (1, 0, 2)).reshape(-1)

    p8_flat = _tc_proj(node_features, W1).reshape(2 * NG * N, GC)
    wc = W1[2 * D:]
    b1_2d = b1.reshape(1, H)

    g0 = _sc_gather(p8_flat, half_idx(0))
    hm0 = _tc_gelu(g0.reshape(NJOB, NGH, CH2, GC), edge_features,
                   wc[:, :NGH * GC], b1_2d[:, :NGH * GC])
    g1 = _sc_gather(p8_flat, half_idx(1))
    s0 = _sc_scatter(hm0.reshape(NGH * E, GC), dst, False)[0]
    hm1 = _tc_gelu(g1.reshape(NJOB, NGH, CH2, GC), edge_features,
                   wc[:, NGH * GC:], b1_2d[:, NGH * GC:])
    s1, deg = _sc_scatter(hm1.reshape(NGH * E, GC), dst, True)

    out = _tc_update(node_features, s0.reshape(NGH, N, GC),
                     s1.reshape(NGH, N, GC), deg.reshape(2, N, GC),
                     W2, b2.reshape(1, D), Wu1, bu1.reshape(1, H),
                     Wu2, bu2.reshape(1, D), gamma.reshape(1, D),
                     beta.reshape(1, D))
    return out
